# baseline (speedup 1.0000x reference)
import jax
import jax.numpy as jnp
from jax.experimental import pallas as pl

_NLAYER = 3
_L = 24
_LPAD = 32
_NTILE = 8


def _gnn_body(xn_ref, adj_ref, ws_ref, wn_ref, b_ref, u_ref, v_ref):
    A = adj_ref[...]
    Xn = xn_ref[...]
    Ab = (A != 0).astype(jnp.float32)
    deg = jnp.sum(Ab, axis=0)
    deg_inv = jnp.where(deg > 0, 1.0 / jnp.maximum(deg, 1.0), 0.0)
    A_s = Ab * deg_inv[None, :]

    def dotT(L, H):
        return jax.lax.dot_general(
            L, H, (((0,), (0,)), ((), ())), preferred_element_type=jnp.float32)

    def lin(H, W):
        cols = [
            jnp.dot(H[:, 128 * t:128 * (t + 1)], W,
                    preferred_element_type=jnp.float32)
            for t in range(_NTILE)
        ]
        return jnp.concatenate(cols, axis=1)

    U = 4.0 * Xn
    V = dotT(A, Xn)
    for i in range(_NLAYER):
        Ws = ws_ref[i]
        Wn = wn_ref[i]
        b = b_ref[i]
        AU = dotT(A_s, U)
        AV = dotT(A_s, V)
        U = lin(U, Ws) + lin(AU, Wn) + b[None, :]
        V = lin(V, Ws) + lin(AV, Wn) + b[None, :]
    u_ref[...] = U
    v_ref[...] = V


def kernel(x, adj, W_self, b_self, W_neigh):
    nS, nC, nN, L = x.shape
    nG = nC * nS
    fpad = nG * _LPAD

    Xn = jnp.transpose(x, (2, 1, 0, 3))
    Xn = jnp.pad(Xn, ((0, 0), (0, 0), (0, 0), (0, _LPAD - L)))
    Xn = Xn.reshape(nN, fpad)

    def mk_tiles(W):
        Wp = jnp.pad(jnp.swapaxes(W, 1, 2),
                     ((0, 0), (0, _LPAD - L), (0, _LPAD - L)))
        z = jnp.zeros_like(Wp)
        rows = [jnp.concatenate([Wp if c == r else z for c in range(4)], axis=2)
                for r in range(4)]
        return jnp.concatenate(rows, axis=1)

    Wst = mk_tiles(W_self)
    Wnt = mk_tiles(W_neigh)
    bt = jnp.tile(jnp.pad(b_self, ((0, 0), (0, _LPAD - L))), (1, nG))

    U, V = pl.pallas_call(
        _gnn_body,
        out_shape=(
            jax.ShapeDtypeStruct((nN, fpad), jnp.float32),
            jax.ShapeDtypeStruct((nN, fpad), jnp.float32),
        ),
    )(Xn, adj, Wst, Wnt, bt)

    Ur = U.reshape(nN, nC, nS, _LPAD)[..., :L].transpose(2, 1, 0, 3)
    Vr = V.reshape(nN, nC, nS, _LPAD)[..., :L].transpose(2, 1, 0, 3)
    out = jnp.stack([Ur, Vr], axis=2).reshape(nS, 2 * nC, nN, L)
    return out
